# baseline (speedup 1.0000x reference)
import functools

import jax
import jax.numpy as jnp
from jax.experimental import pallas as pl
from jax.experimental.pallas import tpu as pltpu


def _tm_kernel(inv_ref, x_ref, w_ref, gr_ref, gc_ref, out_ref):
    x = x_ref[:]
    w = w_ref[:]
    inv = inv_ref[0, 0]

    xw = jax.lax.dot_general(
        x, w, (((1,), (0,)), ((), ())),
        precision=jax.lax.Precision.HIGHEST,
        preferred_element_type=jnp.float32,
    )
    x2 = jnp.sum(x * x, axis=1, keepdims=True)
    w2 = jnp.sum(w * w, axis=0, keepdims=True)
    n2 = x2 - 2.0 * xw + w2

    mn = jnp.min(n2, axis=1, keepdims=True)
    colid = jax.lax.broadcasted_iota(jnp.int32, n2.shape, 1)
    idx = jnp.min(jnp.where(n2 == mn, colid, n2.shape[1]), axis=1,
                  keepdims=True)
    idxf = idx.astype(jnp.float32)

    rowf = gr_ref[:, :]
    colf = gc_ref[:, :]
    side = int(round(float(n2.shape[1]) ** 0.5))
    brow = (idx // side).astype(jnp.float32)
    bcol = (idx % side).astype(jnp.float32)
    dr = rowf - brow
    dc = colf - bcol
    phi = jnp.exp(-inv * (dr * dr + dc * dc))
    recip = 1.0 / jnp.sum(phi, axis=1, keepdims=True)
    out_ref[:] = n2 * (phi * recip)


def kernel(x, std, weights):
    B, D = x.shape
    O = weights.shape[1]
    side = int(round(float(O) ** 0.5))
    BB = 256 if B % 256 == 0 else B

    std_f = jnp.asarray(std).astype(jnp.float32)
    inv = (0.5 * std_f ** (-2)).reshape(1, 1)
    oid = jnp.arange(O, dtype=jnp.int32)
    gr = (oid // side).astype(jnp.float32).reshape(1, O)
    gc = (oid % side).astype(jnp.float32).reshape(1, O)

    return pl.pallas_call(
        _tm_kernel,
        grid=(B // BB,),
        in_specs=[
            pl.BlockSpec(memory_space=pltpu.SMEM),
            pl.BlockSpec((BB, D), lambda i: (i, 0)),
            pl.BlockSpec((D, O), lambda i: (0, 0)),
            pl.BlockSpec((1, O), lambda i: (0, 0)),
            pl.BlockSpec((1, O), lambda i: (0, 0)),
        ],
        out_specs=pl.BlockSpec((BB, O), lambda i: (i, 0)),
        out_shape=jax.ShapeDtypeStruct((B, O), jnp.float32),
    )(inv, x, weights, gr, gc)


# speedup vs baseline: 1.0606x; 1.0606x over previous
import functools

import jax
import jax.numpy as jnp
from jax.experimental import pallas as pl
from jax.experimental.pallas import tpu as pltpu


def _tm_kernel(side, std_ref, x_ref, w_ref, out_ref):
    x = x_ref[:]
    w = w_ref[:]
    s = std_ref[0, 0].astype(jnp.float32)
    inv = 0.5 / (s * s)

    xw = jax.lax.dot_general(
        x, w, (((1,), (0,)), ((), ())),
        precision=jax.lax.Precision.HIGHEST,
        preferred_element_type=jnp.float32,
    )
    x2 = jnp.sum(x * x, axis=1, keepdims=True)
    w2 = jnp.sum(w * w, axis=0, keepdims=True)
    n2 = x2 - 2.0 * xw + w2

    mn = jnp.min(n2, axis=1, keepdims=True)
    colid = jax.lax.broadcasted_iota(jnp.int32, n2.shape, 1)
    idx = jnp.min(jnp.where(n2 == mn, colid, n2.shape[1]), axis=1,
                  keepdims=True)

    rowf = (idx // side).astype(jnp.float32)
    colf = (idx % side).astype(jnp.float32)
    gr = (colid // side).astype(jnp.float32)
    gc = (colid % side).astype(jnp.float32)
    dr = gr - rowf
    dc = gc - colf
    phi = jnp.exp(-inv * (dr * dr + dc * dc))
    recip = 1.0 / jnp.sum(phi, axis=1, keepdims=True)
    out_ref[:] = n2 * (phi * recip)


def kernel(x, std, weights):
    B, D = x.shape
    O = weights.shape[1]
    side = int(round(float(O) ** 0.5))
    BB = 256 if B % 256 == 0 else B

    std2d = jnp.reshape(jnp.asarray(std), (1, 1))
    body = functools.partial(_tm_kernel, side)
    return pl.pallas_call(
        body,
        grid=(B // BB,),
        in_specs=[
            pl.BlockSpec(memory_space=pltpu.SMEM),
            pl.BlockSpec((BB, D), lambda i: (i, 0)),
            pl.BlockSpec((D, O), lambda i: (0, 0)),
        ],
        out_specs=pl.BlockSpec((BB, O), lambda i: (i, 0)),
        out_shape=jax.ShapeDtypeStruct((B, O), jnp.float32),
    )(std2d, x, weights)
